# baseline (speedup 1.0000x reference)
import jax
import jax.numpy as jnp
from jax.experimental import pallas as pl
from jax.experimental.pallas import tpu as pltpu


_NSRC = 8


def _body(col_t_ref, row_t_ref, o_ref, plane_ref, sems):
    col_t = col_t_ref[...]
    row_t = row_t_ref[...]
    d, w = col_t.shape
    h = row_t.shape[1]
    B = o_ref.shape[0]
    col_b = jnp.broadcast_to(col_t[:, None, :], (d, h, w)).reshape(d, h * w)
    row_b = jnp.broadcast_to(row_t[:, :, None], (d, h, w)).reshape(d, h * w)
    for i in range(_NSRC):
        plane_ref[i, 0:d] = col_b
        plane_ref[i, d : 2 * d] = row_b
    copies = [
        pltpu.make_async_copy(plane_ref.at[b % _NSRC], o_ref.at[b], sems.at[b])
        for b in range(B)
    ]
    for c in copies:
        c.start()
    for c in copies:
        c.wait()


def kernel(mask, row_embed, col_embed):
    B = mask.shape[0]
    h, w = mask.shape[-2], mask.shape[-1]
    d = col_embed.shape[-1]
    col_t = col_embed.T
    row_t = row_embed.T

    out = pl.pallas_call(
        _body,
        in_specs=[
            pl.BlockSpec(memory_space=pltpu.MemorySpace.VMEM),
            pl.BlockSpec(memory_space=pltpu.MemorySpace.VMEM),
        ],
        out_specs=pl.BlockSpec(memory_space=pl.ANY),
        out_shape=jax.ShapeDtypeStruct((B, 2 * d, h * w), jnp.float32),
        scratch_shapes=[
            pltpu.VMEM((_NSRC, 2 * d, h * w), jnp.float32),
            pltpu.SemaphoreType.DMA((B,)),
        ],
    )(col_t, row_t)
    return out.reshape(B, 2 * d, h, w)
